# baseline (speedup 1.0000x reference)
import functools

import jax
import jax.numpy as jnp
from jax import lax
from jax.experimental import pallas as pl
from jax.experimental.pallas import tpu as pltpu
from jax.experimental.pallas import tpu_sc as plsc

NC, NS = 2, 16
NW = NC * NS
CHUNK = 128


def kernel(input, weight):
    B0, B1 = input.shape
    D = weight.shape[1]
    N = B0 * B1
    n_chunks = N // CHUNK
    chunks_per_w = n_chunks // NW
    D2 = 2 * D

    flat = input.reshape(N).astype(jnp.int32)
    slab_idx = (flat >> 1).reshape(n_chunks, CHUNK)
    parity = (flat & 1).astype(jnp.bool_)
    slabs = weight.reshape(weight.shape[0] // 2, D2)

    mesh = plsc.VectorSubcoreMesh(core_axis_name="c", subcore_axis_name="s")

    @functools.partial(
        pl.kernel,
        mesh=mesh,
        out_type=jax.ShapeDtypeStruct((N, D2), jnp.float32),
        scratch_types=[
            pltpu.VMEM((chunks_per_w, CHUNK), jnp.int32),
            pltpu.VMEM((CHUNK, D2), jnp.float32),
            pltpu.SemaphoreType.DMA,
        ],
    )
    def sc_gather(table_hbm, idx_hbm, out_hbm, idx_v, rows_v, sem):
        wid = lax.axis_index("s") * NC + lax.axis_index("c")
        crow = wid * chunks_per_w
        pltpu.sync_copy(idx_hbm.at[pl.ds(crow, chunks_per_w)], idx_v)

        @pl.loop(0, chunks_per_w)
        def _(j):
            pltpu.async_copy(table_hbm.at[idx_v.at[j]], rows_v, sem).wait()
            pltpu.sync_copy(rows_v, out_hbm.at[pl.ds((crow + j) * CHUNK, CHUNK)])

    wide = sc_gather(slabs, slab_idx)
    out = jnp.where(parity[:, None], wide[:, D:], wide[:, :D])
    return out.reshape(B0, B1, D)


# speedup vs baseline: 1.3122x; 1.3122x over previous
import functools

import jax
import jax.numpy as jnp
from jax import lax
from jax.experimental import pallas as pl
from jax.experimental.pallas import tpu as pltpu
from jax.experimental.pallas import tpu_sc as plsc

NC, NS = 2, 16
NW = NC * NS
CHUNK = 128


def kernel(input, weight):
    B0, B1 = input.shape
    D = weight.shape[1]
    N = B0 * B1
    n_chunks = N // CHUNK
    chunks_per_w = n_chunks // NW
    D2 = 2 * D

    idx = input.reshape(n_chunks, CHUNK).astype(jnp.int32)
    wide_table = jnp.pad(weight, ((0, 0), (0, D2 - D)))

    mesh = plsc.VectorSubcoreMesh(core_axis_name="c", subcore_axis_name="s")

    @functools.partial(
        pl.kernel,
        mesh=mesh,
        out_type=jax.ShapeDtypeStruct((N, D2), jnp.float32),
        scratch_types=[
            pltpu.VMEM((chunks_per_w, CHUNK), jnp.int32),
            pltpu.VMEM((CHUNK, D2), jnp.float32),
            pltpu.SemaphoreType.DMA,
        ],
    )
    def sc_gather(table_hbm, idx_hbm, out_hbm, idx_v, rows_v, sem):
        wid = lax.axis_index("s") * NC + lax.axis_index("c")
        crow = wid * chunks_per_w
        pltpu.sync_copy(idx_hbm.at[pl.ds(crow, chunks_per_w)], idx_v)

        @pl.loop(0, chunks_per_w)
        def _(j):
            pltpu.async_copy(table_hbm.at[idx_v.at[j]], rows_v, sem).wait()
            pltpu.sync_copy(rows_v, out_hbm.at[pl.ds((crow + j) * CHUNK, CHUNK)])

    wide = sc_gather(wide_table, idx)
    return wide[:, :D].reshape(B0, B1, D)
